# initial kernel scaffold (unmeasured)
import jax
import jax.numpy as jnp
from jax import lax
from jax.experimental import pallas as pl
from jax.experimental.pallas import tpu as pltpu

N_DEV = 16
SQ = 2048
SKV_LOC = 2048
HQ = 8
DH = 128
DM = 1024
CHUNK = SQ // N_DEV
BLK = 64
SCALE = 0.08838834764831843
NHOP = N_DEV - 1


def kernel(x, Wq, K_ext, V_ext, Wo):
    xb = x[0].astype(jnp.bfloat16)
    wqb = Wq.astype(jnp.bfloat16)
    kb = K_ext[0].astype(jnp.bfloat16)
    vb = V_ext[0].astype(jnp.bfloat16)
    wob = Wo.astype(jnp.bfloat16)

    def body(x_ref, wq_ref, k_ref, v_ref, wo_ref, out_ref,
             q_scr, ml_scr, o_scr, recv_ml, recv_o,
             ss_ml, rs_ml, ss_o, rs_o, ss_ag, rs_ag):
        d = lax.axis_index("i")
        left = lax.rem(d + N_DEV - 1, N_DEV)
        right = lax.rem(d + 1, N_DEV)

        barrier = pltpu.get_barrier_semaphore()
        for nbr in (left, right):
            pl.semaphore_signal(barrier, inc=1, device_id=(nbr,),
                                device_id_type=pl.DeviceIdType.MESH)
        pl.semaphore_wait(barrier, 2)

        q = lax.dot_general(x_ref[...], wq_ref[...], (((1,), (0,)), ((), ())),
                            preferred_element_type=jnp.float32)
        q_scr[...] = q.astype(jnp.bfloat16)

        row_i = lax.broadcasted_iota(jnp.int32, (SQ, SKV_LOC), 0)
        col_i = lax.broadcasted_iota(jnp.int32, (SQ, SKV_LOC), 1)
        qb_idx = row_i // BLK
        kb_idx = col_i // BLK + d * (SKV_LOC // BLK)
        mask = (qb_idx == kb_idx) | (kb_idx == 0) | (lax.rem(qb_idx + kb_idx, 3) == 0)

        for h in range(HQ):
            sl = slice(h * DH, (h + 1) * DH)
            s = lax.dot_general(q_scr[:, sl], k_ref[:, h, :],
                                (((1,), (1,)), ((), ())),
                                preferred_element_type=jnp.float32) * SCALE
            s = jnp.where(mask, s, -1e9)
            m_h = jnp.max(s, axis=1, keepdims=True)
            p = jnp.exp(s - m_h)
            l_h = jnp.sum(p, axis=1, keepdims=True)
            o_h = lax.dot_general(p.astype(jnp.bfloat16), v_ref[:, h, :],
                                  (((1,), (0,)), ((), ())),
                                  preferred_element_type=jnp.float32)
            ml_scr[:, h:h + 1] = m_h
            ml_scr[:, HQ + h:HQ + h + 1] = l_h
            o_scr[:, sl] = o_h

        for hop in range(NHOP):
            c_send = lax.rem(d + N_DEV - hop - 1, N_DEV)
            r0 = c_send * CHUNK
            rdma_ml = pltpu.make_async_remote_copy(
                src_ref=ml_scr.at[pl.ds(r0, CHUNK), :],
                dst_ref=recv_ml.at[hop],
                send_sem=ss_ml.at[hop], recv_sem=rs_ml.at[hop],
                device_id=(right,), device_id_type=pl.DeviceIdType.MESH)
            rdma_o = pltpu.make_async_remote_copy(
                src_ref=o_scr.at[pl.ds(r0, CHUNK), :],
                dst_ref=recv_o.at[hop],
                send_sem=ss_o.at[hop], recv_sem=rs_o.at[hop],
                device_id=(right,), device_id_type=pl.DeviceIdType.MESH)
            rdma_ml.start()
            rdma_o.start()
            rdma_ml.wait()
            rdma_o.wait()

            c_recv = lax.rem(d + 2 * N_DEV - hop - 2, N_DEV)
            rr = c_recv * CHUNK
            ml_loc = ml_scr[pl.ds(rr, CHUNK), :]
            ml_rem = recv_ml[hop]
            m_loc, l_loc = ml_loc[:, :HQ], ml_loc[:, HQ:]
            m_rem, l_rem = ml_rem[:, :HQ], ml_rem[:, HQ:]
            m_new = jnp.maximum(m_loc, m_rem)
            a = jnp.exp(m_loc - m_new)
            b = jnp.exp(m_rem - m_new)
            ml_scr[pl.ds(rr, CHUNK), :HQ] = m_new
            ml_scr[pl.ds(rr, CHUNK), HQ:] = a * l_loc + b * l_rem
            for h in range(HQ):
                sl = slice(h * DH, (h + 1) * DH)
                o_scr[pl.ds(rr, CHUNK), sl] = (
                    o_scr[pl.ds(rr, CHUNK), sl] * a[:, h:h + 1]
                    + recv_o[hop][:, sl] * b[:, h:h + 1])

        my0 = d * CHUNK
        l_fin = ml_scr[pl.ds(my0, CHUNK), HQ:]
        acc = jnp.zeros((CHUNK, DM), jnp.float32)
        for h in range(HQ):
            sl = slice(h * DH, (h + 1) * DH)
            ctx_h = (o_scr[pl.ds(my0, CHUNK), sl] / l_fin[:, h:h + 1])
            acc = acc + lax.dot_general(
                ctx_h.astype(jnp.bfloat16), wo_ref[sl, :],
                (((1,), (0,)), ((), ())),
                preferred_element_type=jnp.float32)
        out_ref[pl.ds(my0, CHUNK), :] = acc

        for hop in range(NHOP):
            c_send = lax.rem(d + N_DEV - hop, N_DEV)
            sr = c_send * CHUNK
            rdma = pltpu.make_async_remote_copy(
                src_ref=out_ref.at[pl.ds(sr, CHUNK), :],
                dst_ref=out_ref.at[pl.ds(sr, CHUNK), :],
                send_sem=ss_ag.at[hop], recv_sem=rs_ag.at[hop],
                device_id=(right,), device_id_type=pl.DeviceIdType.MESH)
            rdma.start()
            rdma.wait()

    out = pl.pallas_call(
        body,
        out_shape=jax.ShapeDtypeStruct((SQ, DM), jnp.float32),
        in_specs=[pl.BlockSpec(memory_space=pltpu.VMEM)] * 5,
        out_specs=pl.BlockSpec(memory_space=pltpu.VMEM),
        scratch_shapes=[
            pltpu.VMEM((SQ, DM), jnp.bfloat16),
            pltpu.VMEM((SQ, 2 * HQ), jnp.float32),
            pltpu.VMEM((SQ, DM), jnp.float32),
            pltpu.VMEM((NHOP, CHUNK, 2 * HQ), jnp.float32),
            pltpu.VMEM((NHOP, CHUNK, DM), jnp.float32),
            pltpu.SemaphoreType.DMA((NHOP,)),
            pltpu.SemaphoreType.DMA((NHOP,)),
            pltpu.SemaphoreType.DMA((NHOP,)),
            pltpu.SemaphoreType.DMA((NHOP,)),
            pltpu.SemaphoreType.DMA((NHOP,)),
            pltpu.SemaphoreType.DMA((NHOP,)),
        ],
        compiler_params=pltpu.CompilerParams(
            collective_id=0,
            vmem_limit_bytes=128 * 1024 * 1024,
        ),
    )(xb, wqb, kb, vb, wob)
    return out.reshape(1, SQ, DM)


# baseline (device time: 539500 ns/iter reference)
import jax
import jax.numpy as jnp
from jax import lax
from jax.experimental import pallas as pl
from jax.experimental.pallas import tpu as pltpu

N_DEV = 16
SQ = 2048
SKV_LOC = 2048
HQ = 8
DH = 128
DM = 1024
CHUNK = SQ // N_DEV
BLK = 64
SCALE = 0.08838834764831843
NHOP = N_DEV - 1


def kernel(x, Wq, K_ext, V_ext, Wo):
    xb = x[0].astype(jnp.bfloat16)
    wqb = Wq.astype(jnp.bfloat16)
    kb = K_ext[0].astype(jnp.bfloat16)
    vb = V_ext[0].astype(jnp.bfloat16)
    wob = Wo.astype(jnp.bfloat16)

    def body(x_ref, wq_ref, k_ref, v_ref, wo_ref, out_ref,
             q_scr, ml_scr, o_scr, recv_ml, recv_o,
             ss_ml, rs_ml, ss_o, rs_o, ss_ag, rs_ag):
        d = lax.axis_index("i")
        left = lax.rem(d + N_DEV - 1, N_DEV)
        right = lax.rem(d + 1, N_DEV)

        barrier = pltpu.get_barrier_semaphore()
        for nbr in (left, right):
            pl.semaphore_signal(barrier, inc=1, device_id=(nbr,),
                                device_id_type=pl.DeviceIdType.MESH)
        pl.semaphore_wait(barrier, 2)

        q = lax.dot_general(x_ref[...], wq_ref[...], (((1,), (0,)), ((), ())),
                            preferred_element_type=jnp.float32)
        q_scr[...] = q.astype(jnp.bfloat16)

        QT = 256
        col_i = lax.broadcasted_iota(jnp.int32, (QT, SKV_LOC), 1)
        kb_idx = col_i // BLK + d * (SKV_LOC // BLK)
        row_base = lax.broadcasted_iota(jnp.int32, (QT, SKV_LOC), 0)
        for h in range(HQ):
            sl = slice(h * DH, (h + 1) * DH)

            def tile_body(t, _, sl=sl, h=h):
                r0 = t * QT
                qb_idx = (row_base + r0) // BLK
                mask = ((qb_idx == kb_idx) | (kb_idx == 0)
                        | (lax.rem(qb_idx + kb_idx, 3) == 0))
                s = lax.dot_general(q_scr[pl.ds(r0, QT), sl], k_ref[:, h, :],
                                    (((1,), (1,)), ((), ())),
                                    preferred_element_type=jnp.float32) * SCALE
                s = jnp.where(mask, s, -1e9)
                m_h = jnp.max(s, axis=1, keepdims=True)
                p = jnp.exp(s - m_h)
                l_h = jnp.sum(p, axis=1, keepdims=True)
                o_h = lax.dot_general(p.astype(jnp.bfloat16), v_ref[:, h, :],
                                      (((1,), (0,)), ((), ())),
                                      preferred_element_type=jnp.float32)
                ml_scr[pl.ds(r0, QT), h:h + 1] = m_h
                ml_scr[pl.ds(r0, QT), HQ + h:HQ + h + 1] = l_h
                o_scr[pl.ds(r0, QT), sl] = o_h
                return 0

            lax.fori_loop(0, SQ // QT, tile_body, 0)

        for hop in range(NHOP):
            c_send = lax.rem(d + N_DEV - hop - 1, N_DEV)
            r0 = c_send * CHUNK
            rdma_ml = pltpu.make_async_remote_copy(
                src_ref=ml_scr.at[pl.ds(r0, CHUNK), :],
                dst_ref=recv_ml.at[hop],
                send_sem=ss_ml.at[hop], recv_sem=rs_ml.at[hop],
                device_id=(right,), device_id_type=pl.DeviceIdType.MESH)
            rdma_o = pltpu.make_async_remote_copy(
                src_ref=o_scr.at[pl.ds(r0, CHUNK), :],
                dst_ref=recv_o.at[hop],
                send_sem=ss_o.at[hop], recv_sem=rs_o.at[hop],
                device_id=(right,), device_id_type=pl.DeviceIdType.MESH)
            rdma_ml.start()
            rdma_o.start()
            rdma_ml.wait()
            rdma_o.wait()

            c_recv = lax.rem(d + 2 * N_DEV - hop - 2, N_DEV)
            rr = c_recv * CHUNK
            ml_loc = ml_scr[pl.ds(rr, CHUNK), :]
            ml_rem = recv_ml[hop]
            m_loc, l_loc = ml_loc[:, :HQ], ml_loc[:, HQ:]
            m_rem, l_rem = ml_rem[:, :HQ], ml_rem[:, HQ:]
            m_new = jnp.maximum(m_loc, m_rem)
            a = jnp.exp(m_loc - m_new)
            b = jnp.exp(m_rem - m_new)
            ml_scr[pl.ds(rr, CHUNK), :HQ] = m_new
            ml_scr[pl.ds(rr, CHUNK), HQ:] = a * l_loc + b * l_rem
            for h in range(HQ):
                sl = slice(h * DH, (h + 1) * DH)
                o_scr[pl.ds(rr, CHUNK), sl] = (
                    o_scr[pl.ds(rr, CHUNK), sl] * a[:, h:h + 1]
                    + recv_o[hop][:, sl] * b[:, h:h + 1])

        my0 = d * CHUNK
        l_fin = ml_scr[pl.ds(my0, CHUNK), HQ:]
        acc = jnp.zeros((CHUNK, DM), jnp.float32)
        for h in range(HQ):
            sl = slice(h * DH, (h + 1) * DH)
            ctx_h = (o_scr[pl.ds(my0, CHUNK), sl] / l_fin[:, h:h + 1])
            acc = acc + lax.dot_general(
                ctx_h.astype(jnp.bfloat16), wo_ref[sl, :],
                (((1,), (0,)), ((), ())),
                preferred_element_type=jnp.float32)
        out_ref[pl.ds(my0, CHUNK), :] = acc

        for hop in range(NHOP):
            c_send = lax.rem(d + N_DEV - hop, N_DEV)
            sr = c_send * CHUNK
            rdma = pltpu.make_async_remote_copy(
                src_ref=out_ref.at[pl.ds(sr, CHUNK), :],
                dst_ref=out_ref.at[pl.ds(sr, CHUNK), :],
                send_sem=ss_ag.at[hop], recv_sem=rs_ag.at[hop],
                device_id=(right,), device_id_type=pl.DeviceIdType.MESH)
            rdma.start()
            rdma.wait()

    out = pl.pallas_call(
        body,
        out_shape=jax.ShapeDtypeStruct((SQ, DM), jnp.float32),
        in_specs=[pl.BlockSpec(memory_space=pltpu.VMEM)] * 5,
        out_specs=pl.BlockSpec(memory_space=pltpu.VMEM),
        scratch_shapes=[
            pltpu.VMEM((SQ, DM), jnp.bfloat16),
            pltpu.VMEM((SQ, 2 * HQ), jnp.float32),
            pltpu.VMEM((SQ, DM), jnp.float32),
            pltpu.VMEM((NHOP, CHUNK, 2 * HQ), jnp.float32),
            pltpu.VMEM((NHOP, CHUNK, DM), jnp.float32),
            pltpu.SemaphoreType.DMA((NHOP,)),
            pltpu.SemaphoreType.DMA((NHOP,)),
            pltpu.SemaphoreType.DMA((NHOP,)),
            pltpu.SemaphoreType.DMA((NHOP,)),
            pltpu.SemaphoreType.DMA((NHOP,)),
            pltpu.SemaphoreType.DMA((NHOP,)),
        ],
        compiler_params=pltpu.CompilerParams(
            collective_id=0,
            vmem_limit_bytes=128 * 1024 * 1024,
        ),
    )(xb, wqb, kb, vb, wob)
    return out.reshape(1, SQ, DM)


# device time: 506593 ns/iter; 1.0650x vs baseline; 1.0650x over previous
import jax
import jax.numpy as jnp
from jax import lax
from jax.experimental import pallas as pl
from jax.experimental.pallas import tpu as pltpu

N_DEV = 16
SQ = 2048
SKV_LOC = 2048
HQ = 8
DH = 128
DM = 1024
CHUNK = SQ // N_DEV
BLK = 64
SCALE = 0.08838834764831843
NHOP = N_DEV - 1


def kernel(x, Wq, K_ext, V_ext, Wo):
    xb = x[0].astype(jnp.bfloat16)
    wqb = Wq.astype(jnp.bfloat16)
    kb = K_ext[0].astype(jnp.bfloat16)
    vb = V_ext[0].astype(jnp.bfloat16)
    wob = Wo.astype(jnp.bfloat16)

    def body(x_ref, wq_ref, k_ref, v_ref, wo_ref, out_ref,
             q_scr, ml_scr, o_scr, send_o, recv_ml, recv_o,
             ss_ml, rs_ml, ss_o, rs_o, ss_ag, rs_ag):
        ctx_scr = q_scr
        d = lax.axis_index("i")
        left = lax.rem(d + N_DEV - 1, N_DEV)
        right = lax.rem(d + 1, N_DEV)

        barrier = pltpu.get_barrier_semaphore()
        for nbr in (left, right):
            pl.semaphore_signal(barrier, inc=1, device_id=(nbr,),
                                device_id_type=pl.DeviceIdType.MESH)
        pl.semaphore_wait(barrier, 2)

        q = lax.dot_general(x_ref[...], wq_ref[...], (((1,), (0,)), ((), ())),
                            preferred_element_type=jnp.float32)
        q_scr[...] = q.astype(jnp.bfloat16)

        QT = 128
        col_i = lax.broadcasted_iota(jnp.int32, (QT, SKV_LOC), 1)
        kb_idx = col_i // BLK + d * (SKV_LOC // BLK)
        row_base = lax.broadcasted_iota(jnp.int32, (QT, SKV_LOC), 0)

        def tile_body(t, _):
            r0 = t * QT
            qb_idx = (row_base + r0) // BLK
            mask = ((qb_idx == kb_idx) | (kb_idx == 0)
                    | (lax.rem(qb_idx + kb_idx, 3) == 0))
            neg = jnp.where(mask, 0.0, -1e9)
            for h in range(HQ):
                sl = slice(h * DH, (h + 1) * DH)
                s = lax.dot_general(q_scr[pl.ds(r0, QT), sl], k_ref[:, h, :],
                                    (((1,), (1,)), ((), ())),
                                    preferred_element_type=jnp.float32)
                s = s * SCALE + neg
                m_h = jnp.max(s, axis=1, keepdims=True)
                p = jnp.exp(s - m_h)
                l_h = jnp.sum(p, axis=1, keepdims=True)
                o_h = lax.dot_general(p.astype(jnp.bfloat16), v_ref[:, h, :],
                                      (((1,), (0,)), ((), ())),
                                      preferred_element_type=jnp.float32)
                ml_scr[pl.ds(r0, QT), h:h + 1] = m_h
                ml_scr[pl.ds(r0, QT), HQ + h:HQ + h + 1] = l_h
                o_scr[pl.ds(r0, QT), sl] = o_h
            return 0

        lax.fori_loop(0, SQ // QT, tile_body, 0)

        for hop in range(NHOP):
            c_send = lax.rem(d + N_DEV - hop - 1, N_DEV)
            r0 = c_send * CHUNK
            send_o[...] = o_scr[pl.ds(r0, CHUNK), :].astype(jnp.bfloat16)
            rdma_ml = pltpu.make_async_remote_copy(
                src_ref=ml_scr.at[pl.ds(r0, CHUNK), :],
                dst_ref=recv_ml.at[hop],
                send_sem=ss_ml.at[hop], recv_sem=rs_ml.at[hop],
                device_id=(right,), device_id_type=pl.DeviceIdType.MESH)
            rdma_o = pltpu.make_async_remote_copy(
                src_ref=send_o,
                dst_ref=recv_o.at[hop],
                send_sem=ss_o.at[hop], recv_sem=rs_o.at[hop],
                device_id=(right,), device_id_type=pl.DeviceIdType.MESH)
            rdma_ml.start()
            rdma_o.start()
            rdma_ml.wait()
            rdma_o.wait()

            c_recv = lax.rem(d + 2 * N_DEV - hop - 2, N_DEV)
            rr = c_recv * CHUNK
            ml_loc = ml_scr[pl.ds(rr, CHUNK), :]
            ml_rem = recv_ml[hop]
            m_loc, l_loc = ml_loc[:, :HQ], ml_loc[:, HQ:]
            m_rem, l_rem = ml_rem[:, :HQ], ml_rem[:, HQ:]
            m_new = jnp.maximum(m_loc, m_rem)
            a = jnp.exp(m_loc - m_new)
            b = jnp.exp(m_rem - m_new)
            ml_scr[pl.ds(rr, CHUNK), :HQ] = m_new
            ml_scr[pl.ds(rr, CHUNK), HQ:] = a * l_loc + b * l_rem
            for h in range(HQ):
                sl = slice(h * DH, (h + 1) * DH)
                o_scr[pl.ds(rr, CHUNK), sl] = (
                    o_scr[pl.ds(rr, CHUNK), sl] * a[:, h:h + 1]
                    + recv_o[hop][:, sl].astype(jnp.float32) * b[:, h:h + 1])

        my0 = d * CHUNK
        l_fin = ml_scr[pl.ds(my0, CHUNK), HQ:]
        for h in range(HQ):
            sl = slice(h * DH, (h + 1) * DH)
            ctx_scr[pl.ds(my0, CHUNK), sl] = (
                o_scr[pl.ds(my0, CHUNK), sl] / l_fin[:, h:h + 1]
            ).astype(jnp.bfloat16)

        for hop in range(NHOP):
            c_send = lax.rem(d + N_DEV - hop, N_DEV)
            sr = c_send * CHUNK
            rdma = pltpu.make_async_remote_copy(
                src_ref=ctx_scr.at[pl.ds(sr, CHUNK), :],
                dst_ref=ctx_scr.at[pl.ds(sr, CHUNK), :],
                send_sem=ss_ag.at[hop], recv_sem=rs_ag.at[hop],
                device_id=(right,), device_id_type=pl.DeviceIdType.MESH)
            rdma.start()
            rdma.wait()

        OT = 512
        for t in range(SQ // OT):
            out_ref[t * OT:(t + 1) * OT, :] = lax.dot_general(
                ctx_scr[t * OT:(t + 1) * OT, :], wo_ref[...],
                (((1,), (0,)), ((), ())),
                preferred_element_type=jnp.float32)

    out = pl.pallas_call(
        body,
        out_shape=jax.ShapeDtypeStruct((SQ, DM), jnp.float32),
        in_specs=[pl.BlockSpec(memory_space=pltpu.VMEM)] * 5,
        out_specs=pl.BlockSpec(memory_space=pltpu.VMEM),
        scratch_shapes=[
            pltpu.VMEM((SQ, DM), jnp.bfloat16),
            pltpu.VMEM((SQ, 2 * HQ), jnp.float32),
            pltpu.VMEM((SQ, DM), jnp.float32),
            pltpu.VMEM((CHUNK, DM), jnp.bfloat16),
            pltpu.VMEM((NHOP, CHUNK, 2 * HQ), jnp.float32),
            pltpu.VMEM((NHOP, CHUNK, DM), jnp.bfloat16),
            pltpu.SemaphoreType.DMA((NHOP,)),
            pltpu.SemaphoreType.DMA((NHOP,)),
            pltpu.SemaphoreType.DMA((NHOP,)),
            pltpu.SemaphoreType.DMA((NHOP,)),
            pltpu.SemaphoreType.DMA((NHOP,)),
            pltpu.SemaphoreType.DMA((NHOP,)),
        ],
        compiler_params=pltpu.CompilerParams(
            collective_id=0,
            vmem_limit_bytes=128 * 1024 * 1024,
        ),
    )(xb, wqb, kb, vb, wob)
    return out.reshape(1, SQ, DM)
